# baseline (speedup 1.0000x reference)
import functools

import jax
import jax.numpy as jnp
from jax import lax
from jax.experimental import pallas as pl
from jax.experimental.pallas import tpu as pltpu
from jax.experimental.pallas import tpu_sc as plsc

_TC_IMGS = 8
_SUBROWS = 32
_CHUNK = 2048
_NBINS = 1024


def _argmax_sub(ref, r0, sr):
    c = ref.shape[1]
    val = ref[0, 0, pl.ds(r0, sr), :]
    idx = jnp.zeros(val.shape, dtype=jnp.int32)
    for k in range(1, c):
        cur = ref[0, k, pl.ds(r0, sr), :]
        gt = cur > val
        val = jnp.maximum(cur, val)
        idx = jnp.where(gt, jnp.int32(k), idx)
    return idx


def _tc_body(out_ref, tgt_ref, acc_ref, *, cls_num):
    step = pl.program_id(0) * pl.num_programs(1) + pl.program_id(1)

    @pl.when(step == 0)
    def _init():
        acc_ref[...] = jnp.zeros_like(acc_ref)

    blk_r = out_ref.shape[2]
    tp = [jnp.int32(0)] * cls_num
    co = [jnp.int32(0)] * cls_num
    ct = [jnp.int32(0)] * cls_num
    for r0 in range(0, blk_r, _SUBROWS):
        oi = _argmax_sub(out_ref, r0, _SUBROWS)
        ti = _argmax_sub(tgt_ref, r0, _SUBROWS)
        for c in range(cls_num):
            mo = oi == c
            mt = ti == c
            tp[c] = tp[c] + jnp.sum(mo & mt)
            co[c] = co[c] + jnp.sum(mo)
            ct[c] = ct[c] + jnp.sum(mt)

    rows = lax.broadcasted_iota(jnp.int32, acc_ref.shape, 0)
    lanes = lax.broadcasted_iota(jnp.int32, acc_ref.shape, 1)
    upd = jnp.zeros(acc_ref.shape, dtype=jnp.float32)
    for c in range(cls_num):
        at_c = lanes == c
        upd = upd + jnp.where((rows == 0) & at_c, tp[c].astype(jnp.float32), 0.0)
        upd = upd + jnp.where((rows == 1) & at_c, co[c].astype(jnp.float32), 0.0)
        upd = upd + jnp.where((rows == 2) & at_c, ct[c].astype(jnp.float32), 0.0)
    acc_ref[...] += upd


def _sc_argmax16(buf, off, cls_num):
    val = buf[0, pl.ds(off, 16)]
    idx = jnp.zeros((16,), dtype=jnp.int32)
    for k in range(1, cls_num):
        cur = buf[k, pl.ds(off, 16)]
        gt = cur > val
        val = jnp.where(gt, cur, val)
        idx = jnp.where(gt, jnp.int32(k), idx)
    return idx


def _sc_kernel_body(o_hbm, t_hbm, out_hbm, obuf, tbuf, ones_row,
                    jvec, jsh, dsem, ssem, *keybufs,
                    cls_num, img0, px_per_img):
    cid = lax.axis_index("c")
    sid = lax.axis_index("s")
    img = img0 + cid
    px_per_tile = px_per_img // 16
    n_chunks = px_per_tile // _CHUNK
    base = sid * px_per_tile

    def _fill_ones(i, _i):
        ones_row[pl.ds(i * 16, 16)] = jnp.ones((16,), jnp.float32)
        return _i
    lax.fori_loop(0, 8, _fill_ones, None)

    def _fill_zero(i, _):
        jvec[pl.ds(i * 16, 16)] = jnp.zeros((16,), jnp.float32)
        return _
    lax.fori_loop(0, _NBINS // 16, _fill_zero, None)

    @pl.when(sid == 0)
    def _zero_shared():
        pltpu.sync_copy(jvec, jsh)

    plsc.subcore_barrier()

    def _chunk(ch, _):
        pbase = base + ch * _CHUNK
        cp_o = pltpu.make_async_copy(
            o_hbm.at[img, :, pl.ds(pbase, _CHUNK)], obuf, dsem)
        cp_t = pltpu.make_async_copy(
            t_hbm.at[img, :, pl.ds(pbase, _CHUNK)], tbuf, dsem)
        cp_o.start()
        cp_t.start()
        cp_o.wait()
        cp_t.wait()

        for r in range(16):
            kb_row = keybufs[r]

            def _vec(c8, _i, kb_row=kb_row, r=r):
                off = r * 128 + c8 * 16
                oi = _sc_argmax16(obuf, off, cls_num)
                ti = _sc_argmax16(tbuf, off, cls_num)
                kb_row[pl.ds(c8 * 16, 16)] = oi * cls_num + ti
                return _i
            lax.fori_loop(0, 8, _vec, None)
            pltpu.async_copy(ones_row, jsh.at[kb_row], ssem, add=True)
        for r in range(16):
            pltpu.make_async_copy(ones_row, jsh.at[keybufs[r]], ssem).wait()
        return _
    lax.fori_loop(0, n_chunks, _chunk, None)

    plsc.subcore_barrier()

    @pl.when(sid == 0)
    def _writeback():
        pltpu.sync_copy(jsh, out_hbm.at[cid])


def _final_body(tc_ref, jm_ref, jmt_ref, score_ref, *, cls_num):
    jm = jm_ref[...]
    jmt = jmt_ref[...]
    rows = lax.broadcasted_iota(jnp.int32, jm.shape, 0)
    lanes = lax.broadcasted_iota(jnp.int32, jm.shape, 1)
    diag = jnp.where(rows == lanes, jm, 0.0)
    sc_co = jnp.sum(jmt, axis=0, keepdims=True)
    sc_ct = jnp.sum(jm, axis=0, keepdims=True)
    sc_tp = jnp.sum(diag, axis=0, keepdims=True)

    acc = tc_ref[...]
    tps = acc[0:1, :] + 0.0 * sc_tp
    denom = acc[1:2, :] + 0.0 * sc_co + acc[2:3, :] + 0.0 * sc_ct - tps
    iou = jnp.where(denom > 0.0, tps / denom, 0.0)
    score_ref[...] = jnp.sum(iou, keepdims=True) / jnp.float32(cls_num)


def kernel(output, target):
    b, c, h, w = output.shape
    blk_r = 256
    n_r = h // blk_r

    tc_counts = pl.pallas_call(
        functools.partial(_tc_body, cls_num=c),
        grid=(_TC_IMGS, n_r),
        in_specs=[pl.BlockSpec((1, c, blk_r, w), lambda i, r: (i, 0, r, 0))] * 2,
        out_specs=pl.BlockSpec((3, 32), lambda i, r: (0, 0)),
        out_shape=jax.ShapeDtypeStruct((3, 32), jnp.float32),
    )(output, target)

    px = h * w
    o_flat = output.reshape(b, c, px)
    t_flat = target.reshape(b, c, px)
    mesh = plsc.VectorSubcoreMesh(core_axis_name="c", subcore_axis_name="s")
    sc_counts = pl.kernel(
        functools.partial(
            _sc_kernel_body, cls_num=c, img0=6, px_per_img=px),
        mesh=mesh,
        out_type=jax.ShapeDtypeStruct((2, _NBINS), jnp.float32),
        scratch_types=[
            pltpu.VMEM((c, _CHUNK), jnp.float32),
            pltpu.VMEM((c, _CHUNK), jnp.float32),
            pltpu.VMEM((128,), jnp.float32),
            pltpu.VMEM((_NBINS,), jnp.float32),
            pltpu.VMEM_SHARED((_NBINS,), jnp.float32),
            pltpu.SemaphoreType.DMA,
            pltpu.SemaphoreType.DMA,
        ] + [pltpu.VMEM((128,), jnp.int32) for _ in range(16)],
    )(o_flat, t_flat)

    jtot = (sc_counts[0] + sc_counts[1])[: c * c].reshape(c, c)
    jm = jnp.pad(jtot, ((0, 0), (0, 32 - c)))
    jmt = jnp.pad(jtot.T, ((0, 0), (0, 32 - c)))

    score = pl.pallas_call(
        functools.partial(_final_body, cls_num=c),
        in_specs=[
            pl.BlockSpec((3, 32), lambda: (0, 0)),
            pl.BlockSpec((c, 32), lambda: (0, 0)),
            pl.BlockSpec((c, 32), lambda: (0, 0)),
        ],
        out_specs=pl.BlockSpec((1, 1), lambda: (0, 0)),
        out_shape=jax.ShapeDtypeStruct((1, 1), jnp.float32),
    )(tc_counts, jm, jmt)
    return score[0, 0]


# speedup vs baseline: 4.7678x; 4.7678x over previous
import functools

import jax
import jax.numpy as jnp
from jax import lax
from jax.experimental import pallas as pl
from jax.experimental.pallas import tpu as pltpu

_SUBROWS = 32


def _argmax_sub(ref, r0, sr):
    c = ref.shape[1]
    val = ref[0, 0, pl.ds(r0, sr), :]
    idx = jnp.zeros(val.shape, dtype=jnp.int32)
    for k in range(1, c):
        cur = ref[0, k, pl.ds(r0, sr), :]
        gt = cur > val
        val = jnp.maximum(cur, val)
        idx = jnp.where(gt, jnp.int32(k), idx)
    return idx


def _iou_body(out_ref, tgt_ref, score_ref, acc_ref, *, nsteps, cls_num):
    step = pl.program_id(0) * pl.num_programs(1) + pl.program_id(1)

    @pl.when(step == 0)
    def _init():
        for j in range(3):
            for c in range(cls_num):
                acc_ref[j, c] = jnp.int32(0)

    blk_r = out_ref.shape[2]
    tp = [jnp.int32(0)] * cls_num
    co = [jnp.int32(0)] * cls_num
    ct = [jnp.int32(0)] * cls_num
    for r0 in range(0, blk_r, _SUBROWS):
        oi = _argmax_sub(out_ref, r0, _SUBROWS)
        ti = _argmax_sub(tgt_ref, r0, _SUBROWS)
        for c in range(cls_num):
            mo = oi == c
            mt = ti == c
            tp[c] = tp[c] + jnp.sum(mo & mt)
            co[c] = co[c] + jnp.sum(mo)
            ct[c] = ct[c] + jnp.sum(mt)

    for c in range(cls_num):
        acc_ref[0, c] = acc_ref[0, c] + tp[c]
        acc_ref[1, c] = acc_ref[1, c] + co[c]
        acc_ref[2, c] = acc_ref[2, c] + ct[c]

    @pl.when(step == nsteps - 1)
    def _finish():
        shape = (3, 32)
        rows = lax.broadcasted_iota(jnp.int32, shape, 0)
        lanes = lax.broadcasted_iota(jnp.int32, shape, 1)
        acc = jnp.zeros(shape, dtype=jnp.float32)
        for c in range(cls_num):
            at_c = lanes == c
            for j in range(3):
                acc = acc + jnp.where(
                    (rows == j) & at_c, acc_ref[j, c].astype(jnp.float32), 0.0)
        tps = acc[0:1, :]
        denom = acc[1:2, :] + acc[2:3, :] - tps
        iou = jnp.where(denom > 0.0, tps / denom, 0.0)
        score_ref[...] = jnp.sum(iou, keepdims=True) / jnp.float32(cls_num)


def kernel(output, target):
    b, c, h, w = output.shape
    blk_r = 256
    n_r = h // blk_r
    nsteps = b * n_r

    body = functools.partial(_iou_body, nsteps=nsteps, cls_num=c)
    in_spec = pl.BlockSpec((1, c, blk_r, w), lambda i, r: (i, 0, r, 0))
    score = pl.pallas_call(
        body,
        grid=(b, n_r),
        in_specs=[in_spec, in_spec],
        out_specs=pl.BlockSpec((1, 1), lambda i, r: (0, 0)),
        out_shape=jax.ShapeDtypeStruct((1, 1), jnp.float32),
        scratch_shapes=[pltpu.SMEM((3, c), jnp.int32)],
    )(output, target)
    return score[0, 0]


# speedup vs baseline: 4.9247x; 1.0329x over previous
import functools

import jax
import jax.numpy as jnp
from jax.experimental import pallas as pl
from jax.experimental.pallas import tpu as pltpu

_SUBROWS = 32


def _argmax_sub(ref, r0, sr):
    c = ref.shape[1]
    val = ref[0, 0, pl.ds(r0, sr), :]
    idx = jnp.zeros(val.shape, dtype=jnp.int32)
    for k in range(1, c):
        cur = ref[0, k, pl.ds(r0, sr), :]
        gt = cur > val
        val = jnp.maximum(cur, val)
        idx = jnp.where(gt, jnp.int32(k), idx)
    return idx


def _iou_body(out_ref, out2_ref, tgt_ref, tgt2_ref, score_ref, acc_ref, *,
              nsteps, cls_num):
    step = pl.program_id(0) * pl.num_programs(1) + pl.program_id(1)

    @pl.when(step == 0)
    def _init():
        acc_ref[...] = jnp.zeros_like(acc_ref)

    tp = [jnp.int32(0)] * cls_num
    co = [jnp.int32(0)] * cls_num
    ct = [jnp.int32(0)] * cls_num
    subtiles = [(ref, r0) for ref in (out_ref, out2_ref, tgt_ref, tgt2_ref)
                for r0 in range(0, ref.shape[2], _SUBROWS)]
    half = len(subtiles) // 2
    for (oref, r0), (tref, t0) in zip(subtiles[:half], subtiles[half:]):
        oi = _argmax_sub(oref, r0, _SUBROWS)
        ti = _argmax_sub(tref, t0, _SUBROWS)
        for c in range(cls_num):
            mo = oi == c
            mt = ti == c
            tp[c] = tp[c] + jnp.sum(mo & mt)
            co[c] = co[c] + jnp.sum(mo)
            ct[c] = ct[c] + jnp.sum(mt)

    rows = jax.lax.broadcasted_iota(jnp.int32, acc_ref.shape, 0)
    lanes = jax.lax.broadcasted_iota(jnp.int32, acc_ref.shape, 1)
    upd = jnp.zeros(acc_ref.shape, dtype=jnp.float32)
    for c in range(cls_num):
        at_c = lanes == c
        upd = upd + jnp.where((rows == 0) & at_c, tp[c].astype(jnp.float32), 0.0)
        upd = upd + jnp.where((rows == 1) & at_c, co[c].astype(jnp.float32), 0.0)
        upd = upd + jnp.where((rows == 2) & at_c, ct[c].astype(jnp.float32), 0.0)
    acc_ref[...] += upd

    @pl.when(step == nsteps - 1)
    def _finish():
        acc = acc_ref[...]
        tps = acc[0:1, :]
        denom = acc[1:2, :] + acc[2:3, :] - tps
        iou = jnp.where(denom > 0.0, tps / denom, 0.0)
        score_ref[...] = jnp.sum(iou, keepdims=True) / jnp.float32(cls_num)


def kernel(output, target):
    b, c, h, w = output.shape
    blk_r = 256
    n_r = h // blk_r
    nsteps = b * n_r

    body = functools.partial(_iou_body, nsteps=nsteps, cls_num=c)
    spec_lo = pl.BlockSpec((1, c, blk_r // 2, w), lambda i, r: (i, 0, 2 * r, 0))
    spec_hi = pl.BlockSpec(
        (1, c, blk_r // 2, w), lambda i, r: (i, 0, 2 * r + 1, 0))
    score = pl.pallas_call(
        body,
        grid=(b, n_r),
        in_specs=[spec_lo, spec_hi, spec_lo, spec_hi],
        out_specs=pl.BlockSpec((1, 1), lambda i, r: (0, 0)),
        out_shape=jax.ShapeDtypeStruct((1, 1), jnp.float32),
        scratch_shapes=[pltpu.VMEM((3, 32), jnp.float32)],
    )(output, output, target, target)
    return score[0, 0]


# speedup vs baseline: 4.9787x; 1.0110x over previous
import functools

import jax
import jax.numpy as jnp
from jax.experimental import pallas as pl
from jax.experimental.pallas import tpu as pltpu

_SUBROWS = 32


def _argmax_sub(ref, r0, sr):
    c = ref.shape[1]
    val = ref[0, 0, pl.ds(r0, sr), :]
    idx = jnp.zeros(val.shape, dtype=jnp.int32)
    for k in range(1, c):
        cur = ref[0, k, pl.ds(r0, sr), :]
        gt = cur > val
        val = jnp.maximum(cur, val)
        idx = jnp.where(gt, jnp.int32(k), idx)
    return idx


def _iou_body(*refs, nsteps, cls_num):
    (o1, o2, o3, o4, t1, t2, t3, t4, score_ref, acc_ref) = refs
    step = pl.program_id(0) * pl.num_programs(1) + pl.program_id(1)

    @pl.when(step == 0)
    def _init():
        acc_ref[...] = jnp.zeros_like(acc_ref)

    tp = [jnp.int32(0)] * cls_num
    co = [jnp.int32(0)] * cls_num
    ct = [jnp.int32(0)] * cls_num
    subtiles = [(ref, r0) for ref in (o1, o2, o3, o4, t1, t2, t3, t4)
                for r0 in range(0, ref.shape[2], _SUBROWS)]
    half = len(subtiles) // 2
    for (oref, r0), (tref, t0) in zip(subtiles[:half], subtiles[half:]):
        oi = _argmax_sub(oref, r0, _SUBROWS)
        ti = _argmax_sub(tref, t0, _SUBROWS)
        for c in range(cls_num):
            mo = oi == c
            mt = ti == c
            tp[c] = tp[c] + jnp.sum(mo & mt)
            co[c] = co[c] + jnp.sum(mo)
            ct[c] = ct[c] + jnp.sum(mt)

    rows = jax.lax.broadcasted_iota(jnp.int32, acc_ref.shape, 0)
    lanes = jax.lax.broadcasted_iota(jnp.int32, acc_ref.shape, 1)
    upd = jnp.zeros(acc_ref.shape, dtype=jnp.float32)
    for c in range(cls_num):
        at_c = lanes == c
        upd = upd + jnp.where((rows == 0) & at_c, tp[c].astype(jnp.float32), 0.0)
        upd = upd + jnp.where((rows == 1) & at_c, co[c].astype(jnp.float32), 0.0)
        upd = upd + jnp.where((rows == 2) & at_c, ct[c].astype(jnp.float32), 0.0)
    acc_ref[...] += upd

    @pl.when(step == nsteps - 1)
    def _finish():
        acc = acc_ref[...]
        tps = acc[0:1, :]
        denom = acc[1:2, :] + acc[2:3, :] - tps
        iou = jnp.where(denom > 0.0, tps / denom, 0.0)
        score_ref[...] = jnp.sum(iou, keepdims=True) / jnp.float32(cls_num)


def kernel(output, target):
    b, c, h, w = output.shape
    blk_r = 256
    n_r = h // blk_r
    nsteps = b * n_r

    body = functools.partial(_iou_body, nsteps=nsteps, cls_num=c)
    specs = [
        pl.BlockSpec((1, c, blk_r // 4, w),
                     functools.partial(lambda q, i, r: (i, 0, 4 * r + q, 0), q))
        for q in range(4)
    ]
    score = pl.pallas_call(
        body,
        grid=(b, n_r),
        in_specs=specs + specs,
        out_specs=pl.BlockSpec((1, 1), lambda i, r: (0, 0)),
        out_shape=jax.ShapeDtypeStruct((1, 1), jnp.float32),
        scratch_shapes=[pltpu.VMEM((3, 32), jnp.float32)],
    )(output, output, output, output, target, target, target, target)
    return score[0, 0]
